# baseline (speedup 1.0000x reference)
import functools

import jax
import jax.numpy as jnp
import numpy as np
from jax import lax
from jax.experimental import pallas as pl
from jax.experimental.pallas import tpu as pltpu
from jax.experimental.pallas import tpu_sc as plsc

_NF = 26
_VOCAB = 1000
_EMB = 64
_B = 4096
_H1 = 1024
_H2 = 512
_DIN = _NF * _EMB

_NC = 2
_NS = 16
_NW = _NC * _NS
_R = _B * _NF
_ROWS_PER_W = _R // _NW
_CHUNK = 128
_NCHUNK = _ROWS_PER_W // _CHUNK

_NT = 8
_BT = _B // _NT


def _dst_perm():
    b = np.arange(_B)[:, None]
    f = np.arange(_NF)[None, :]
    d = (b // 8) * 208 + (f // 2) * 16 + (b % 8) * 2 + (f % 2)
    return jnp.asarray(d.reshape(_NW, _NCHUNK, _CHUNK), dtype=jnp.int32)


@functools.lru_cache(maxsize=None)
def _make_sc_gather():
    mesh = plsc.VectorSubcoreMesh(core_axis_name="c", subcore_axis_name="s")

    @functools.partial(
        pl.kernel,
        out_type=jax.ShapeDtypeStruct((_R, _EMB), jnp.float32),
        mesh=mesh,
        scratch_types=[
            pltpu.VMEM((_NCHUNK, _CHUNK), jnp.int32),
            pltpu.VMEM((_NCHUNK, _CHUNK), jnp.int32),
            pltpu.VMEM((4, _CHUNK, _EMB), jnp.float32),
            pltpu.SemaphoreType.DMA,
            pltpu.SemaphoreType.DMA,
        ],
        compiler_params=pltpu.CompilerParams(use_tc_tiling_on_sc=False),
    )
    def _sc_gather(tab_hbm, idx_hbm, dst_hbm, out_hbm, idx_v, dst_v, rows_v,
                   gsem, ssem):
        wid = lax.axis_index("s") * _NC + lax.axis_index("c")
        pltpu.sync_copy(idx_hbm.at[wid], idx_v)
        pltpu.sync_copy(dst_hbm.at[wid], dst_v)

        def gather(c):
            return pltpu.async_copy(tab_hbm.at[idx_v.at[c]], rows_v.at[c % 4], gsem)

        def scatter(c):
            return pltpu.async_copy(rows_v.at[c % 4], out_hbm.at[dst_v.at[c]], ssem)

        g_d = {0: gather(0), 1: gather(1)}
        s_d = {}
        for c in range(_NCHUNK):
            if c >= 2:
                s_d[c - 2].wait()
            if c + 2 < _NCHUNK:
                g_d[c + 2] = gather(c + 2)
            g_d[c].wait()
            s_d[c] = scatter(c)
        s_d[_NCHUNK - 2].wait()
        s_d[_NCHUNK - 1].wait()

    return _sc_gather


def _mlp_body(emb_ref, w1_ref, b1_ref, g1_ref, be1_ref, w2_ref, b2_ref, g2_ref,
              be2_ref, w3_ref, b3_ref, out_ref, h1_ref, h2_ref, s1_ref, q1_ref,
              w1c_ref, w2c_ref):
    t = pl.program_id(0)

    @pl.when(t == 0)
    def _cast_weights():
        w1c_ref[...] = w1_ref[...].astype(jnp.bfloat16)
        w2c_ref[...] = w2_ref[...].astype(jnp.bfloat16)
    e4 = emb_ref[...].reshape(_BT // 8, 13, 8, 128)
    h1_t = b1_ref[...]
    for k in range(13):
        a_k = e4[:, k, :, :].reshape(_BT, 128).astype(jnp.bfloat16)
        h1_t = h1_t + jnp.dot(a_k, w1c_ref[pl.ds(128 * k, 128), :],
                              preferred_element_type=jnp.float32)
    h1_ref[pl.ds(t * _BT, _BT), :] = h1_t.astype(jnp.bfloat16)
    s = jnp.sum(h1_t, axis=0, keepdims=True)
    q = jnp.sum(h1_t * h1_t, axis=0, keepdims=True)

    @pl.when(t == 0)
    def _init():
        s1_ref[...] = s
        q1_ref[...] = q

    @pl.when(t > 0)
    def _accum():
        s1_ref[...] += s
        q1_ref[...] += q

    @pl.when(t == _NT - 1)
    def _finish():
        n = float(_B)
        eps = 1e-5

        mean1 = s1_ref[...] / n
        var1 = q1_ref[...] / n - mean1 * mean1
        scale1 = g1_ref[...] * lax.rsqrt(var1 + eps)
        shift1 = be1_ref[...] - mean1 * scale1

        s2 = jnp.zeros((1, _H2), jnp.float32)
        q2 = jnp.zeros((1, _H2), jnp.float32)
        for i in range(_NT):
            h1_i = h1_ref[pl.ds(i * _BT, _BT), :].astype(jnp.float32)
            a = jnp.maximum(h1_i * scale1 + shift1, 0.0)
            h2_t = (
                jnp.dot(a.astype(jnp.bfloat16), w2c_ref[...],
                        preferred_element_type=jnp.float32)
                + b2_ref[...]
            )
            h2_ref[pl.ds(i * _BT, _BT), :] = h2_t
            s2 = s2 + jnp.sum(h2_t, axis=0, keepdims=True)
            q2 = q2 + jnp.sum(h2_t * h2_t, axis=0, keepdims=True)

        mean2 = s2 / n
        var2 = q2 / n - mean2 * mean2
        scale2 = g2_ref[...] * lax.rsqrt(var2 + eps)
        shift2 = be2_ref[...] - mean2 * scale2

        for i in range(_NT):
            a2 = jnp.maximum(h2_ref[pl.ds(i * _BT, _BT), :] * scale2 + shift2, 0.0)
            z = jnp.sum(a2 * w3_ref[...], axis=1) + b3_ref[0, 0]
            out_ref[pl.ds(i * _BT, _BT)] = 1.0 / (1.0 + jnp.exp(-z))


_mlp = pl.pallas_call(
    _mlp_body,
    grid=(_NT,),
    in_specs=[
        pl.BlockSpec((_BT * _DIN,), lambda t: (t,)),
        pl.BlockSpec((_DIN, _H1), lambda t: (0, 0)),
        pl.BlockSpec((1, _H1), lambda t: (0, 0)),
        pl.BlockSpec((1, _H1), lambda t: (0, 0)),
        pl.BlockSpec((1, _H1), lambda t: (0, 0)),
        pl.BlockSpec((_H1, _H2), lambda t: (0, 0)),
        pl.BlockSpec((1, _H2), lambda t: (0, 0)),
        pl.BlockSpec((1, _H2), lambda t: (0, 0)),
        pl.BlockSpec((1, _H2), lambda t: (0, 0)),
        pl.BlockSpec((1, _H2), lambda t: (0, 0)),
        pl.BlockSpec((1, 1), lambda t: (0, 0)),
    ],
    out_specs=pl.BlockSpec((_B,), lambda t: (0,)),
    out_shape=jax.ShapeDtypeStruct((_B,), jnp.float32),
    scratch_shapes=[
        pltpu.VMEM((_B, _H1), jnp.bfloat16),
        pltpu.VMEM((_B, _H2), jnp.float32),
        pltpu.VMEM((1, _H1), jnp.float32),
        pltpu.VMEM((1, _H1), jnp.float32),
        pltpu.VMEM((_DIN, _H1), jnp.bfloat16),
        pltpu.VMEM((_H1, _H2), jnp.bfloat16),
    ],
    compiler_params=pltpu.CompilerParams(dimension_semantics=("arbitrary",)),
)


def kernel(field_0, field_1, field_2, field_3, field_4, field_5, field_6,
           field_7, field_8, field_9, field_10, field_11, field_12, field_13,
           field_14, field_15, field_16, field_17, field_18, field_19,
           field_20, field_21, field_22, field_23, field_24, field_25,
           tables, W1, b1, g1, be1, W2, b2, g2, be2, W3, b3):
    fields = (field_0, field_1, field_2, field_3, field_4, field_5, field_6,
              field_7, field_8, field_9, field_10, field_11, field_12,
              field_13, field_14, field_15, field_16, field_17, field_18,
              field_19, field_20, field_21, field_22, field_23, field_24,
              field_25)
    idx = jnp.stack(fields, axis=1).astype(jnp.int32)
    idx = idx + (jnp.arange(_NF, dtype=jnp.int32) * _VOCAB)[None, :]
    idx = idx.reshape(_NW, _NCHUNK, _CHUNK)

    tab_flat = tables.reshape(_NF * _VOCAB, _EMB)
    emb = _make_sc_gather()(tab_flat, idx, _dst_perm()).reshape(_B * _DIN)

    out = _mlp(emb, W1, b1.reshape(1, _H1), g1.reshape(1, _H1),
               be1.reshape(1, _H1), W2, b2.reshape(1, _H2),
               g2.reshape(1, _H2), be2.reshape(1, _H2), W3.reshape(1, _H2),
               b3.reshape(1, 1))
    return out


# speedup vs baseline: 1.0007x; 1.0007x over previous
import functools

import jax
import jax.numpy as jnp
import numpy as np
from jax import lax
from jax.experimental import pallas as pl
from jax.experimental.pallas import tpu as pltpu
from jax.experimental.pallas import tpu_sc as plsc

_NF = 26
_VOCAB = 1000
_EMB = 64
_B = 4096
_H1 = 1024
_H2 = 512
_DIN = _NF * _EMB

_NC = 2
_NS = 16
_NW = _NC * _NS
_R = _B * _NF
_ROWS_PER_W = _R // _NW
_CHUNK = 128
_NCHUNK = _ROWS_PER_W // _CHUNK

_NT = 8
_BT = _B // _NT


def _dst_perm():
    b = np.arange(_B)[:, None]
    f = np.arange(_NF)[None, :]
    d = (b // 8) * 208 + (f // 2) * 16 + (b % 8) * 2 + (f % 2)
    return jnp.asarray(d.reshape(_NW, _NCHUNK, _CHUNK), dtype=jnp.int32)


@functools.lru_cache(maxsize=None)
def _make_sc_gather():
    mesh = plsc.VectorSubcoreMesh(core_axis_name="c", subcore_axis_name="s")

    @functools.partial(
        pl.kernel,
        out_type=jax.ShapeDtypeStruct((_R, _EMB), jnp.float32),
        mesh=mesh,
        scratch_types=[
            pltpu.VMEM((_NCHUNK, _CHUNK), jnp.int32),
            pltpu.VMEM((_NCHUNK, _CHUNK), jnp.int32),
            pltpu.VMEM((6, _CHUNK, _EMB), jnp.float32),
            pltpu.SemaphoreType.DMA,
            pltpu.SemaphoreType.DMA,
        ],
        compiler_params=pltpu.CompilerParams(use_tc_tiling_on_sc=False),
    )
    def _sc_gather(tab_hbm, idx_hbm, dst_hbm, out_hbm, idx_v, dst_v, rows_v,
                   gsem, ssem):
        wid = lax.axis_index("s") * _NC + lax.axis_index("c")
        pltpu.sync_copy(idx_hbm.at[wid], idx_v)
        pltpu.sync_copy(dst_hbm.at[wid], dst_v)

        def gather(c):
            return pltpu.async_copy(tab_hbm.at[idx_v.at[c]], rows_v.at[c % 6], gsem)

        def scatter(c):
            return pltpu.async_copy(rows_v.at[c % 6], out_hbm.at[dst_v.at[c]], ssem)

        g_d = {c: gather(c) for c in range(3)}
        s_d = {}
        for c in range(_NCHUNK):
            if c >= 3:
                s_d[c - 3].wait()
            if c + 3 < _NCHUNK:
                g_d[c + 3] = gather(c + 3)
            g_d[c].wait()
            s_d[c] = scatter(c)
        for c in range(_NCHUNK - 3, _NCHUNK):
            s_d[c].wait()

    return _sc_gather


def _mlp_body(emb_ref, w1_ref, b1_ref, g1_ref, be1_ref, w2_ref, b2_ref, g2_ref,
              be2_ref, w3_ref, b3_ref, out_ref, h1_ref, h2_ref, s1_ref, q1_ref,
              w1c_ref, w2c_ref):
    t = pl.program_id(0)

    @pl.when(t == 0)
    def _cast_weights():
        w1c_ref[...] = w1_ref[...].astype(jnp.bfloat16)
        w2c_ref[...] = w2_ref[...].astype(jnp.bfloat16)
    e4 = emb_ref[...].reshape(_BT // 8, 13, 8, 128)
    h1_t = b1_ref[...]
    for k in range(13):
        a_k = e4[:, k, :, :].reshape(_BT, 128).astype(jnp.bfloat16)
        h1_t = h1_t + jnp.dot(a_k, w1c_ref[pl.ds(128 * k, 128), :],
                              preferred_element_type=jnp.float32)
    h1_ref[pl.ds(t * _BT, _BT), :] = h1_t
    s = jnp.sum(h1_t, axis=0, keepdims=True)
    q = jnp.sum(h1_t * h1_t, axis=0, keepdims=True)

    @pl.when(t == 0)
    def _init():
        s1_ref[...] = s
        q1_ref[...] = q

    @pl.when(t > 0)
    def _accum():
        s1_ref[...] += s
        q1_ref[...] += q

    @pl.when(t == _NT - 1)
    def _finish():
        n = float(_B)
        eps = 1e-5

        mean1 = s1_ref[...] / n
        var1 = q1_ref[...] / n - mean1 * mean1
        scale1 = g1_ref[...] * lax.rsqrt(var1 + eps)
        shift1 = be1_ref[...] - mean1 * scale1

        s2 = jnp.zeros((1, _H2), jnp.float32)
        q2 = jnp.zeros((1, _H2), jnp.float32)
        for i in range(_NT):
            a = jnp.maximum(h1_ref[pl.ds(i * _BT, _BT), :] * scale1 + shift1, 0.0)
            h2_t = (
                jnp.dot(a.astype(jnp.bfloat16), w2c_ref[...],
                        preferred_element_type=jnp.float32)
                + b2_ref[...]
            )
            h2_ref[pl.ds(i * _BT, _BT), :] = h2_t
            s2 = s2 + jnp.sum(h2_t, axis=0, keepdims=True)
            q2 = q2 + jnp.sum(h2_t * h2_t, axis=0, keepdims=True)

        mean2 = s2 / n
        var2 = q2 / n - mean2 * mean2
        scale2 = g2_ref[...] * lax.rsqrt(var2 + eps)
        shift2 = be2_ref[...] - mean2 * scale2

        for i in range(_NT):
            a2 = jnp.maximum(h2_ref[pl.ds(i * _BT, _BT), :] * scale2 + shift2, 0.0)
            z = jnp.sum(a2 * w3_ref[...], axis=1) + b3_ref[0, 0]
            out_ref[pl.ds(i * _BT, _BT)] = 1.0 / (1.0 + jnp.exp(-z))


_mlp = pl.pallas_call(
    _mlp_body,
    grid=(_NT,),
    in_specs=[
        pl.BlockSpec((_BT * _DIN,), lambda t: (t,)),
        pl.BlockSpec((_DIN, _H1), lambda t: (0, 0)),
        pl.BlockSpec((1, _H1), lambda t: (0, 0)),
        pl.BlockSpec((1, _H1), lambda t: (0, 0)),
        pl.BlockSpec((1, _H1), lambda t: (0, 0)),
        pl.BlockSpec((_H1, _H2), lambda t: (0, 0)),
        pl.BlockSpec((1, _H2), lambda t: (0, 0)),
        pl.BlockSpec((1, _H2), lambda t: (0, 0)),
        pl.BlockSpec((1, _H2), lambda t: (0, 0)),
        pl.BlockSpec((1, _H2), lambda t: (0, 0)),
        pl.BlockSpec((1, 1), lambda t: (0, 0)),
    ],
    out_specs=pl.BlockSpec((_B,), lambda t: (0,)),
    out_shape=jax.ShapeDtypeStruct((_B,), jnp.float32),
    scratch_shapes=[
        pltpu.VMEM((_B, _H1), jnp.float32),
        pltpu.VMEM((_B, _H2), jnp.float32),
        pltpu.VMEM((1, _H1), jnp.float32),
        pltpu.VMEM((1, _H1), jnp.float32),
        pltpu.VMEM((_DIN, _H1), jnp.bfloat16),
        pltpu.VMEM((_H1, _H2), jnp.bfloat16),
    ],
    compiler_params=pltpu.CompilerParams(dimension_semantics=("arbitrary",)),
)


def kernel(field_0, field_1, field_2, field_3, field_4, field_5, field_6,
           field_7, field_8, field_9, field_10, field_11, field_12, field_13,
           field_14, field_15, field_16, field_17, field_18, field_19,
           field_20, field_21, field_22, field_23, field_24, field_25,
           tables, W1, b1, g1, be1, W2, b2, g2, be2, W3, b3):
    fields = (field_0, field_1, field_2, field_3, field_4, field_5, field_6,
              field_7, field_8, field_9, field_10, field_11, field_12,
              field_13, field_14, field_15, field_16, field_17, field_18,
              field_19, field_20, field_21, field_22, field_23, field_24,
              field_25)
    idx = jnp.stack(fields, axis=1).astype(jnp.int32)
    idx = idx + (jnp.arange(_NF, dtype=jnp.int32) * _VOCAB)[None, :]
    idx = idx.reshape(_NW, _NCHUNK, _CHUNK)

    tab_flat = tables.reshape(_NF * _VOCAB, _EMB)
    emb = _make_sc_gather()(tab_flat, idx, _dst_perm()).reshape(_B * _DIN)

    out = _mlp(emb, W1, b1.reshape(1, _H1), g1.reshape(1, _H1),
               be1.reshape(1, _H1), W2, b2.reshape(1, _H2),
               g2.reshape(1, _H2), be2.reshape(1, _H2), W3.reshape(1, _H2),
               b3.reshape(1, 1))
    return out


# speedup vs baseline: 1.0048x; 1.0040x over previous
import functools

import jax
import jax.numpy as jnp
import numpy as np
from jax import lax
from jax.experimental import pallas as pl
from jax.experimental.pallas import tpu as pltpu
from jax.experimental.pallas import tpu_sc as plsc

_NF = 26
_VOCAB = 1000
_EMB = 64
_B = 4096
_H1 = 1024
_H2 = 512
_DIN = _NF * _EMB

_NC = 2
_NS = 16
_NW = _NC * _NS
_R = _B * _NF
_ROWS_PER_W = _R // _NW
_CHUNK = 128
_NCHUNK = _ROWS_PER_W // _CHUNK

_NT = 8
_BT = _B // _NT


def _dst_perm():
    b = np.arange(_B)[:, None]
    f = np.arange(_NF)[None, :]
    d = (b // 8) * 208 + (f // 2) * 16 + (b % 8) * 2 + (f % 2)
    return jnp.asarray(d.reshape(_NW, _NCHUNK, _CHUNK), dtype=jnp.int32)


@functools.lru_cache(maxsize=None)
def _make_sc_gather():
    mesh = plsc.VectorSubcoreMesh(core_axis_name="c", subcore_axis_name="s")

    @functools.partial(
        pl.kernel,
        out_type=jax.ShapeDtypeStruct((_R, _EMB), jnp.float32),
        mesh=mesh,
        scratch_types=[
            pltpu.VMEM((_NCHUNK, _CHUNK), jnp.int32),
            pltpu.VMEM((_NCHUNK, _CHUNK), jnp.int32),
            pltpu.VMEM((4, _CHUNK, _EMB), jnp.float32),
            pltpu.SemaphoreType.DMA,
            pltpu.SemaphoreType.DMA,
        ],
        compiler_params=pltpu.CompilerParams(use_tc_tiling_on_sc=False),
    )
    def _sc_gather(tab_hbm, idx_hbm, dst_hbm, out_hbm, idx_v, dst_v, rows_v,
                   gsem, ssem):
        wid = lax.axis_index("s") * _NC + lax.axis_index("c")
        pltpu.sync_copy(idx_hbm.at[wid], idx_v)
        pltpu.sync_copy(dst_hbm.at[wid], dst_v)

        def gather(c):
            return pltpu.async_copy(tab_hbm.at[idx_v.at[c]], rows_v.at[c % 4], gsem)

        def scatter(c):
            return pltpu.async_copy(rows_v.at[c % 4], out_hbm.at[dst_v.at[c]], ssem)

        g_d = {0: gather(0), 1: gather(1)}
        s_d = {}
        for c in range(_NCHUNK):
            if c >= 2:
                s_d[c - 2].wait()
            if c + 2 < _NCHUNK:
                g_d[c + 2] = gather(c + 2)
            g_d[c].wait()
            s_d[c] = scatter(c)
        s_d[_NCHUNK - 2].wait()
        s_d[_NCHUNK - 1].wait()

    return _sc_gather


def _mlp_body(emb_ref, w1_ref, b1_ref, g1_ref, be1_ref, w2_ref, b2_ref, g2_ref,
              be2_ref, w3_ref, b3_ref, out_ref, h1_ref, h2_ref, s1_ref, q1_ref,
              w1c_ref, w2c_ref):
    t = pl.program_id(0)

    @pl.when(t == 0)
    def _cast_weights():
        w1c_ref[...] = w1_ref[...].astype(jnp.bfloat16)
        w2c_ref[...] = w2_ref[...].astype(jnp.bfloat16)
    e4 = emb_ref[...].reshape(_BT // 8, 13, 8, 128)
    h1_t = b1_ref[...]
    for k in range(13):
        a_k = e4[:, k, :, :].reshape(_BT, 128).astype(jnp.bfloat16)
        h1_t = h1_t + jnp.dot(a_k, w1c_ref[pl.ds(128 * k, 128), :],
                              preferred_element_type=jnp.float32)
    h1_ref[pl.ds(t * _BT, _BT), :] = h1_t
    s = jnp.sum(h1_t, axis=0, keepdims=True)
    q = jnp.sum(h1_t * h1_t, axis=0, keepdims=True)

    @pl.when(t == 0)
    def _init():
        s1_ref[...] = s
        q1_ref[...] = q

    @pl.when(t > 0)
    def _accum():
        s1_ref[...] += s
        q1_ref[...] += q

    @pl.when(t == _NT - 1)
    def _finish():
        n = float(_B)
        eps = 1e-5

        mean1 = s1_ref[...] / n
        var1 = q1_ref[...] / n - mean1 * mean1
        scale1 = g1_ref[...] * lax.rsqrt(var1 + eps)
        shift1 = be1_ref[...] - mean1 * scale1

        s2 = jnp.zeros((1, _H2), jnp.float32)
        q2 = jnp.zeros((1, _H2), jnp.float32)
        for i in range(_NT):
            a = jnp.maximum(h1_ref[pl.ds(i * _BT, _BT), :] * scale1 + shift1, 0.0)
            h2_t = (
                jnp.dot(a.astype(jnp.bfloat16), w2c_ref[...],
                        preferred_element_type=jnp.float32)
                + b2_ref[...]
            )
            h2_ref[pl.ds(i * _BT, _BT), :] = h2_t
            s2 = s2 + jnp.sum(h2_t, axis=0, keepdims=True)
            q2 = q2 + jnp.sum(h2_t * h2_t, axis=0, keepdims=True)

        mean2 = s2 / n
        var2 = q2 / n - mean2 * mean2
        scale2 = g2_ref[...] * lax.rsqrt(var2 + eps)
        shift2 = be2_ref[...] - mean2 * scale2

        for i in range(_NT):
            a2 = jnp.maximum(h2_ref[pl.ds(i * _BT, _BT), :] * scale2 + shift2, 0.0)
            z = jnp.sum(a2 * w3_ref[...], axis=1) + b3_ref[0, 0]
            out_ref[pl.ds(i * _BT, _BT)] = 1.0 / (1.0 + jnp.exp(-z))


_mlp = pl.pallas_call(
    _mlp_body,
    grid=(_NT,),
    in_specs=[
        pl.BlockSpec((_BT * _DIN,), lambda t: (t,)),
        pl.BlockSpec((_DIN, _H1), lambda t: (0, 0)),
        pl.BlockSpec((1, _H1), lambda t: (0, 0)),
        pl.BlockSpec((1, _H1), lambda t: (0, 0)),
        pl.BlockSpec((1, _H1), lambda t: (0, 0)),
        pl.BlockSpec((_H1, _H2), lambda t: (0, 0)),
        pl.BlockSpec((1, _H2), lambda t: (0, 0)),
        pl.BlockSpec((1, _H2), lambda t: (0, 0)),
        pl.BlockSpec((1, _H2), lambda t: (0, 0)),
        pl.BlockSpec((1, _H2), lambda t: (0, 0)),
        pl.BlockSpec((1, 1), lambda t: (0, 0)),
    ],
    out_specs=pl.BlockSpec((_B,), lambda t: (0,)),
    out_shape=jax.ShapeDtypeStruct((_B,), jnp.float32),
    scratch_shapes=[
        pltpu.VMEM((_B, _H1), jnp.float32),
        pltpu.VMEM((_B, _H2), jnp.float32),
        pltpu.VMEM((1, _H1), jnp.float32),
        pltpu.VMEM((1, _H1), jnp.float32),
        pltpu.VMEM((_DIN, _H1), jnp.bfloat16),
        pltpu.VMEM((_H1, _H2), jnp.bfloat16),
    ],
    compiler_params=pltpu.CompilerParams(dimension_semantics=("arbitrary",)),
)


def kernel(field_0, field_1, field_2, field_3, field_4, field_5, field_6,
           field_7, field_8, field_9, field_10, field_11, field_12, field_13,
           field_14, field_15, field_16, field_17, field_18, field_19,
           field_20, field_21, field_22, field_23, field_24, field_25,
           tables, W1, b1, g1, be1, W2, b2, g2, be2, W3, b3):
    fields = (field_0, field_1, field_2, field_3, field_4, field_5, field_6,
              field_7, field_8, field_9, field_10, field_11, field_12,
              field_13, field_14, field_15, field_16, field_17, field_18,
              field_19, field_20, field_21, field_22, field_23, field_24,
              field_25)
    idx = jnp.stack(fields, axis=1).astype(jnp.int32)
    idx = idx + (jnp.arange(_NF, dtype=jnp.int32) * _VOCAB)[None, :]
    idx = idx.reshape(_NW, _NCHUNK, _CHUNK)

    tab_flat = tables.reshape(_NF * _VOCAB, _EMB)
    emb = _make_sc_gather()(tab_flat, idx, _dst_perm()).reshape(_B * _DIN)

    out = _mlp(emb, W1, b1.reshape(1, _H1), g1.reshape(1, _H1),
               be1.reshape(1, _H1), W2, b2.reshape(1, _H2),
               g2.reshape(1, _H2), be2.reshape(1, _H2), W3.reshape(1, _H2),
               b3.reshape(1, 1))
    return out
